# baseline (speedup 1.0000x reference)
import functools

import jax
import jax.numpy as jnp
from jax import lax
from jax.experimental import pallas as pl
from jax.experimental.pallas import tpu as pltpu
from jax.experimental.pallas import tpu_sc as plsc

D_MODEL = 1024
NUM_WAVES = 3
N_TOKENS = 4 * 8192
NUM_CORES = 2
NUM_SUBCORES = 16
NUM_WORKERS = NUM_CORES * NUM_SUBCORES
TOK_PER_W = N_TOKENS // NUM_WORKERS
CHUNK = 32
NCHUNK = TOK_PER_W // CHUNK
LANES = 16

_mesh = plsc.VectorSubcoreMesh(core_axis_name="c", subcore_axis_name="s")


@functools.partial(
    pl.kernel,
    mesh=_mesh,
    out_type=jax.ShapeDtypeStruct((N_TOKENS, D_MODEL), jnp.float32),
    scratch_types=[
        pltpu.VMEM((TOK_PER_W,), jnp.float32),
        pltpu.VMEM((TOK_PER_W,), jnp.float32),
        pltpu.VMEM((TOK_PER_W,), jnp.float32),
        pltpu.VMEM((TOK_PER_W,), jnp.int32),
        pltpu.VMEM((CHUNK, D_MODEL), jnp.float32),
        pltpu.VMEM((CHUNK, D_MODEL), jnp.float32),
        pltpu.SemaphoreType.DMA,
        pltpu.SemaphoreType.DMA,
    ],
)
def _sc_kernel(m0_h, m1_h, m2_h, tab_h, out_h,
               m0_v, m1_v, m2_v, idx_v, buf0, buf1, sem0, sem1):
    wid = lax.axis_index("s") * NUM_CORES + lax.axis_index("c")
    base = wid * TOK_PER_W

    pltpu.sync_copy(m0_h.at[pl.ds(base, TOK_PER_W)], m0_v)
    pltpu.sync_copy(m1_h.at[pl.ds(base, TOK_PER_W)], m1_v)
    pltpu.sync_copy(m2_h.at[pl.ds(base, TOK_PER_W)], m2_v)

    one = jnp.full((LANES,), 1, jnp.int32)
    zero = jnp.full((LANES,), 0, jnp.int32)
    two = jnp.full((LANES,), 2, jnp.int32)

    def label_step(i, carry):
        a0 = m0_v[pl.ds(i * LANES, LANES)]
        a1 = m1_v[pl.ds(i * LANES, LANES)]
        a2 = m2_v[pl.ds(i * LANES, LANES)]
        lbl = jnp.where(a1 > a0, one, zero)
        mx = jnp.maximum(a0, a1)
        lbl = jnp.where(a2 > mx, two, lbl)
        idx_v[pl.ds(i * LANES, LANES)] = lbl
        return carry

    lax.fori_loop(0, TOK_PER_W // LANES, label_step, 0)

    bufs = (buf0, buf1)
    sems = (sem0, sem1)

    def start_gather(k, slot):
        pltpu.async_copy(
            tab_h.at[idx_v.at[pl.ds(k * CHUNK, CHUNK)]], bufs[slot], sems[slot]
        )

    start_gather(0, 0)

    def chunk_step(p, carry):
        k = p * 2
        pltpu.make_async_copy(
            tab_h.at[idx_v.at[pl.ds(k * CHUNK, CHUNK)]], bufs[0], sems[0]
        ).wait()
        start_gather(k + 1, 1)
        pltpu.sync_copy(bufs[0], out_h.at[pl.ds(base + k * CHUNK, CHUNK)])
        pltpu.make_async_copy(
            tab_h.at[idx_v.at[pl.ds((k + 1) * CHUNK, CHUNK)]], bufs[1], sems[1]
        ).wait()

        @pl.when(p + 1 < NCHUNK // 2)
        def _():
            start_gather(k + 2, 0)

        pltpu.sync_copy(bufs[1], out_h.at[pl.ds(base + (k + 1) * CHUNK, CHUNK)])
        return carry

    lax.fori_loop(0, NCHUNK // 2, chunk_step, 0)


def kernel(wave_mask, wave_embedding):
    B, S, W = wave_mask.shape
    maskT = wave_mask.reshape(B * S, W).T
    out = _sc_kernel(maskT[0], maskT[1], maskT[2], wave_embedding)
    return out.reshape(B, S, D_MODEL)


# speedup vs baseline: 3.2685x; 3.2685x over previous
import functools

import jax
import jax.numpy as jnp
from jax import lax
from jax.experimental import pallas as pl
from jax.experimental.pallas import tpu as pltpu
from jax.experimental.pallas import tpu_sc as plsc

D_MODEL = 1024
NUM_WAVES = 3
N_TOKENS = 4 * 8192
NUM_CORES = 2
NUM_SUBCORES = 16
NUM_WORKERS = NUM_CORES * NUM_SUBCORES
TOK_PER_W = N_TOKENS // NUM_WORKERS
CHUNK = 32
NCHUNK = TOK_PER_W // CHUNK
LANES = 16

_mesh = plsc.VectorSubcoreMesh(core_axis_name="c", subcore_axis_name="s")


@functools.partial(
    pl.kernel,
    mesh=_mesh,
    out_type=jax.ShapeDtypeStruct((N_TOKENS, D_MODEL), jnp.float32),
    scratch_types=[
        pltpu.VMEM((TOK_PER_W,), jnp.float32),
        pltpu.VMEM((TOK_PER_W,), jnp.float32),
        pltpu.VMEM((TOK_PER_W,), jnp.float32),
        pltpu.VMEM((TOK_PER_W,), jnp.int32),
        pltpu.VMEM((CHUNK, D_MODEL), jnp.float32),
        pltpu.VMEM((CHUNK, D_MODEL), jnp.float32),
        pltpu.SemaphoreType.DMA,
        pltpu.SemaphoreType.DMA,
    ],
)
def _sc_kernel(m0_h, m1_h, m2_h, tab_h, out_h,
               m0_v, m1_v, m2_v, idx_v, buf0, buf1, sem0, sem1):
    wid = lax.axis_index("s") * NUM_CORES + lax.axis_index("c")
    base = wid * TOK_PER_W

    pltpu.sync_copy(m0_h.at[pl.ds(base, TOK_PER_W)], m0_v)
    pltpu.sync_copy(m1_h.at[pl.ds(base, TOK_PER_W)], m1_v)
    pltpu.sync_copy(m2_h.at[pl.ds(base, TOK_PER_W)], m2_v)

    one = jnp.full((LANES,), 1, jnp.int32)
    zero = jnp.full((LANES,), 0, jnp.int32)
    two = jnp.full((LANES,), 2, jnp.int32)
    tab_off = jnp.full((LANES,), 0, jnp.int32) + wid * NUM_WAVES

    def label_step(i, carry):
        a0 = m0_v[pl.ds(i * LANES, LANES)]
        a1 = m1_v[pl.ds(i * LANES, LANES)]
        a2 = m2_v[pl.ds(i * LANES, LANES)]
        lbl = jnp.where(a1 > a0, one, zero)
        mx = jnp.maximum(a0, a1)
        lbl = jnp.where(a2 > mx, two, lbl)
        idx_v[pl.ds(i * LANES, LANES)] = lbl + tab_off
        return carry

    lax.fori_loop(0, TOK_PER_W // LANES, label_step, 0)

    bufs = (buf0, buf1)
    sems = (sem0, sem1)

    def start_gather(k, slot):
        pltpu.async_copy(
            tab_h.at[idx_v.at[pl.ds(k * CHUNK, CHUNK)]], bufs[slot], sems[slot]
        )

    start_gather(0, 0)

    def chunk_step(p, carry):
        k = p * 2
        pltpu.make_async_copy(
            tab_h.at[idx_v.at[pl.ds(k * CHUNK, CHUNK)]], bufs[0], sems[0]
        ).wait()
        start_gather(k + 1, 1)
        pltpu.sync_copy(bufs[0], out_h.at[pl.ds(base + k * CHUNK, CHUNK)])
        pltpu.make_async_copy(
            tab_h.at[idx_v.at[pl.ds((k + 1) * CHUNK, CHUNK)]], bufs[1], sems[1]
        ).wait()

        @pl.when(p + 1 < NCHUNK // 2)
        def _():
            start_gather(k + 2, 0)

        pltpu.sync_copy(bufs[1], out_h.at[pl.ds(base + (k + 1) * CHUNK, CHUNK)])
        return carry

    lax.fori_loop(0, NCHUNK // 2, chunk_step, 0)


def kernel(wave_mask, wave_embedding):
    B, S, W = wave_mask.shape
    maskT = wave_mask.reshape(B * S, W).T
    tab_rep = jnp.tile(wave_embedding, (NUM_WORKERS, 1))
    out = _sc_kernel(maskT[0], maskT[1], maskT[2], tab_rep)
    return out.reshape(B, S, D_MODEL)
